# baseline (speedup 1.0000x reference)
import jax
import jax.numpy as jnp
from jax.experimental import pallas as pl

N_TOKENS = 32768
D_MODEL = 768
ROUTE_SIZE = 8
TOP_K = 2
BLOCK = 2048
HALF = N_TOKENS // 2


def _topk_softmax(logits, logits_ref, idx_ref, wts_ref):
    logits_ref[...] = logits
    m1 = jnp.max(logits, axis=-1)
    i1 = jnp.argmax(logits, axis=-1).astype(jnp.int32)
    cols = jax.lax.broadcasted_iota(jnp.int32, logits.shape, 1)
    masked = jnp.where(cols == i1[:, None], -jnp.inf, logits)
    m2 = jnp.max(masked, axis=-1)
    i2 = jnp.argmax(masked, axis=-1).astype(jnp.int32)
    idx_ref[...] = jnp.concatenate([i1[:, None], i2[:, None]], axis=-1)
    e2 = jnp.exp(m2 - m1)
    denom = 1.0 + e2
    wts_ref[...] = jnp.concatenate(
        [(1.0 / denom)[:, None], (e2 / denom)[:, None]], axis=-1
    )


def _router_kernel(xa_ref, xb_ref, w_ref,
                   logits_a_ref, idx_a_ref, wts_a_ref,
                   logits_b_ref, idx_b_ref, wts_b_ref):
    w = w_ref[...]
    dims = (((1,), (1,)), ((), ()))
    la = jax.lax.dot_general(xa_ref[...], w, dims,
                             preferred_element_type=jnp.float32)
    _topk_softmax(la, logits_a_ref, idx_a_ref, wts_a_ref)
    lb = jax.lax.dot_general(xb_ref[...], w, dims,
                             preferred_element_type=jnp.float32)
    _topk_softmax(lb, logits_b_ref, idx_b_ref, wts_b_ref)


@jax.jit
def kernel(x, W):
    xa = x[:HALF]
    xb = x[HALF:]
    grid = (HALF // BLOCK,)
    half_shapes = (
        jax.ShapeDtypeStruct((HALF, ROUTE_SIZE), jnp.float32),
        jax.ShapeDtypeStruct((HALF, TOP_K), jnp.int32),
        jax.ShapeDtypeStruct((HALF, TOP_K), jnp.float32),
    )
    out_spec_half = (
        pl.BlockSpec((BLOCK, ROUTE_SIZE), lambda i: (i, 0)),
        pl.BlockSpec((BLOCK, TOP_K), lambda i: (i, 0)),
        pl.BlockSpec((BLOCK, TOP_K), lambda i: (i, 0)),
    )
    la, ia, wa, lb, ib, wb = pl.pallas_call(
        _router_kernel,
        grid=grid,
        in_specs=[
            pl.BlockSpec((BLOCK, D_MODEL), lambda i: (i, 0)),
            pl.BlockSpec((BLOCK, D_MODEL), lambda i: (i, 0)),
            pl.BlockSpec((ROUTE_SIZE, D_MODEL), lambda i: (0, 0)),
        ],
        out_specs=out_spec_half + out_spec_half,
        out_shape=half_shapes + half_shapes,
    )(xa, xb, W)
    logits = jnp.concatenate([la, lb], axis=0)
    idx = jnp.concatenate([ia, ib], axis=0)
    wts = jnp.concatenate([wa, wb], axis=0)
    return idx, wts, logits


# speedup vs baseline: 1.7274x; 1.7274x over previous
import jax
import jax.numpy as jnp
from jax.experimental import pallas as pl

N_TOKENS = 32768
D_MODEL = 768
ROUTE_SIZE = 8
TOP_K = 2
BLOCK = 2048
HALF = N_TOKENS // 2


def _topk_softmax(logits, logits_ref, idx_ref, wts_ref):
    logits_ref[...] = logits
    m1 = jnp.max(logits, axis=-1)
    i1 = jnp.argmax(logits, axis=-1).astype(jnp.int32)
    cols = jax.lax.broadcasted_iota(jnp.int32, logits.shape, 1)
    masked = jnp.where(cols == i1[:, None], -jnp.inf, logits)
    m2 = jnp.max(masked, axis=-1)
    i2 = jnp.argmax(masked, axis=-1).astype(jnp.int32)
    idx_ref[...] = jnp.concatenate([i1[:, None], i2[:, None]], axis=-1)
    e2 = jnp.exp(m2 - m1)
    denom = 1.0 + e2
    wts_ref[...] = jnp.concatenate(
        [(1.0 / denom)[:, None], (e2 / denom)[:, None]], axis=-1
    )


def _router_kernel(xa_ref, xb_ref, w_ref,
                   logits_a_ref, idx_a_ref, wts_a_ref,
                   logits_b_ref, idx_b_ref, wts_b_ref):
    w = w_ref[...]
    dims = (((1,), (1,)), ((), ()))
    la = jax.lax.dot_general(xa_ref[...], w, dims,
                             preferred_element_type=jnp.float32)
    _topk_softmax(la, logits_a_ref, idx_a_ref, wts_a_ref)
    lb = jax.lax.dot_general(xb_ref[...], w, dims,
                             preferred_element_type=jnp.float32)
    _topk_softmax(lb, logits_b_ref, idx_b_ref, wts_b_ref)


@jax.jit
def kernel(x, W):
    grid = (HALF // BLOCK,)
    nb = HALF // BLOCK
    half_shapes = (
        jax.ShapeDtypeStruct((HALF, ROUTE_SIZE), jnp.float32),
        jax.ShapeDtypeStruct((HALF, TOP_K), jnp.int32),
        jax.ShapeDtypeStruct((HALF, TOP_K), jnp.float32),
    )
    out_spec_half = (
        pl.BlockSpec((BLOCK, ROUTE_SIZE), lambda i: (i, 0)),
        pl.BlockSpec((BLOCK, TOP_K), lambda i: (i, 0)),
        pl.BlockSpec((BLOCK, TOP_K), lambda i: (i, 0)),
    )
    la, ia, wa, lb, ib, wb = pl.pallas_call(
        _router_kernel,
        grid=grid,
        in_specs=[
            pl.BlockSpec((BLOCK, D_MODEL), lambda i: (i, 0)),
            pl.BlockSpec((BLOCK, D_MODEL), lambda i: (i + nb, 0)),
            pl.BlockSpec((ROUTE_SIZE, D_MODEL), lambda i: (0, 0)),
        ],
        out_specs=out_spec_half + out_spec_half,
        out_shape=half_shapes + half_shapes,
    )(x, x, W)
    logits = jnp.concatenate([la, lb], axis=0)
    idx = jnp.concatenate([ia, ib], axis=0)
    wts = jnp.concatenate([wa, wb], axis=0)
    return idx, wts, logits
